# Initial kernel scaffold; baseline (speedup 1.0000x reference)
import jax
import jax.numpy as jnp
from jax.experimental import pallas as pl


def kernel(x, edge_index, W_self_0, W_neigh_0, b_0, W_self_1, W_neigh_1, b_1, W_self_2, W_neigh_2, b_2):
    raise NotImplementedError("write your pallas kernel here")


# speedup vs baseline: 6.2085x; 6.2085x over previous
import functools

import jax
import jax.numpy as jnp
from jax import lax
from jax.experimental import pallas as pl
from jax.experimental.pallas import tpu as pltpu
from jax.experimental.pallas import tpu_sc as plsc

_N = 10000
_E = 160000
_D = 256
_H = 128
_NSUB = 16
_K = 80
_EPS = _E // _NSUB
_NCH = _EPS // _K
_ROWS = 624
_TAIL = _ROWS - 7 * _K
_EXTRA = _N - _NSUB * _ROWS

_MESH = plsc.VectorSubcoreMesh(core_axis_name="c", subcore_axis_name="s")


def _zero_slices(zsrc, dst_spmem, rbase, s):
    nz = zsrc.shape[0]
    full, tail = divmod(_ROWS, nz)
    for j in range(full):
        pltpu.sync_copy(zsrc, dst_spmem.at[pl.ds(rbase + j * nz, nz)])
    if tail:
        pltpu.sync_copy(zsrc.at[pl.ds(0, tail)],
                        dst_spmem.at[pl.ds(rbase + full * nz, tail)])

    @pl.when(s == _NSUB - 1)
    def _():
        pltpu.sync_copy(zsrc.at[pl.ds(0, _EXTRA)],
                        dst_spmem.at[pl.ds(_NSUB * _ROWS, _EXTRA)])


def _copy_out_slices(acc_spmem, bounce, out_ref, rbase, s):
    nz = bounce.shape[0]
    full, tail = divmod(_ROWS, nz)
    sizes = [nz] * full + ([tail] if tail else [])
    for j, sz in enumerate(sizes):
        r0 = rbase + j * nz
        pltpu.sync_copy(acc_spmem.at[pl.ds(r0, sz)], bounce.at[pl.ds(0, sz)])
        pltpu.sync_copy(bounce.at[pl.ds(0, sz)], out_ref.at[pl.ds(r0, sz)])

    @pl.when(s == _NSUB - 1)
    def _():
        r0 = _NSUB * _ROWS
        pltpu.sync_copy(acc_spmem.at[pl.ds(r0, _EXTRA)],
                        bounce.at[pl.ds(0, _EXTRA)])
        pltpu.sync_copy(bounce.at[pl.ds(0, _EXTRA)],
                        out_ref.at[pl.ds(r0, _EXTRA)])


def _make_agg():
    out_type = [
        jax.ShapeDtypeStruct((_N, _H), jnp.float32),
        jax.ShapeDtypeStruct((_N, _H), jnp.float32),
    ]
    scratch_types = [
        pltpu.VMEM((2, _K), jnp.int32),
        pltpu.VMEM((2, _K), jnp.int32),
        pltpu.VMEM((_K, _H), jnp.float32),
        pltpu.VMEM((_K, _H), jnp.float32),
        pltpu.VMEM_SHARED((_N, _H), jnp.float32),
        pltpu.SemaphoreType.DMA,
        pltpu.SemaphoreType.DMA,
        pltpu.SemaphoreType.DMA,
        pltpu.SemaphoreType.DMA,
    ]

    @functools.partial(pl.kernel, mesh=_MESH, out_type=out_type,
                       scratch_types=scratch_types)
    def agg(ha, hb, edges, zrows, out_a, out_b, ib0, ib1, st0, st1,
            acc, semi0, semi1, semg0, semg1):
        c = lax.axis_index("c")
        s = lax.axis_index("s")
        rbase = s * _ROWS
        my_edges = edges.at[s]

        pltpu.sync_copy(zrows, st0)
        _zero_slices(st0, acc, rbase, s)

        plsc.subcore_barrier()

        def run(h):
            def idx_load(i, ib, sem):
                pltpu.async_copy(my_edges.at[i], ib, sem)

            def idx_wait(ib, sem):
                pltpu.make_async_copy(my_edges.at[0], ib, sem).wait()

            def gather(ib, buf, sem):
                pltpu.async_copy(h.at[ib.at[0]], buf, sem)

            def gather_wait(ib, buf, sem):
                pltpu.make_async_copy(h.at[ib.at[0]], buf, sem).wait()

            def scat(ib, buf):
                pltpu.sync_copy(buf, acc.at[ib.at[1]], add=True)

            pltpu.sync_copy(my_edges.at[0], ib0)
            idx_load(1, ib1, semi1)
            gather(ib0, st0, semg0)

            def step(j, carry):
                i0 = 2 * j
                idx_wait(ib1, semi1)
                gather(ib1, st1, semg1)
                gather_wait(ib0, st0, semg0)
                scat(ib0, st0)
                idx_load(i0 + 2, ib0, semi0)
                idx_wait(ib0, semi0)
                gather(ib0, st0, semg0)
                gather_wait(ib1, st1, semg1)
                scat(ib1, st1)

                @pl.when(i0 + 3 < _NCH)
                def _():
                    idx_load(i0 + 3, ib1, semi1)

                return carry

            lax.fori_loop(0, (_NCH - 1) // 2, step, 0)
            gather_wait(ib0, st0, semg0)
            scat(ib0, st0)

        @pl.when(c == 0)
        def _():
            run(ha)

        @pl.when(c == 1)
        def _():
            run(hb)

        plsc.subcore_barrier()

        @pl.when(c == 0)
        def _():
            _copy_out_slices(acc, st0, out_a, rbase, s)

        @pl.when(c == 1)
        def _():
            _copy_out_slices(acc, st0, out_b, rbase, s)

    return agg


def _make_deg():
    kd = 40
    nchd = _E // 2 // _NSUB // kd
    out_type = [
        jax.ShapeDtypeStruct((_N, _H), jnp.float32),
        jax.ShapeDtypeStruct((_N, _H), jnp.float32),
    ]
    scratch_types = [
        pltpu.VMEM((kd,), jnp.int32),
        pltpu.VMEM((kd,), jnp.int32),
        pltpu.VMEM((kd, _H), jnp.float32),
        pltpu.VMEM_SHARED((_N, _H), jnp.float32),
        pltpu.SemaphoreType.DMA,
        pltpu.SemaphoreType.DMA,
    ]

    @functools.partial(pl.kernel, mesh=_MESH, out_type=out_type,
                       scratch_types=scratch_types)
    def deg(dst3, zrows, ones_h, deg_a, deg_b, ib0, ib1, st, dacc,
            semi0, semi1):
        c = lax.axis_index("c")
        s = lax.axis_index("s")
        rbase = s * _ROWS
        w = c * _NSUB + s
        my_dst = dst3.at[w]

        pltpu.sync_copy(zrows, st)
        _zero_slices(st, dacc, rbase, s)
        pltpu.sync_copy(ones_h, st)
        plsc.subcore_barrier()

        def idx_load(i, ib, sem):
            pltpu.async_copy(my_dst.at[i], ib, sem)

        def idx_wait(ib, sem):
            pltpu.make_async_copy(my_dst.at[0], ib, sem).wait()

        def scat(ib):
            pltpu.sync_copy(st, dacc.at[ib], add=True)

        pltpu.sync_copy(my_dst.at[0], ib0)
        idx_load(1, ib1, semi1)

        def step(j, carry):
            i0 = 2 * j
            scat(ib0)
            idx_load(i0 + 2, ib0, semi0)
            idx_wait(ib1, semi1)
            scat(ib1)

            @pl.when(i0 + 3 < nchd)
            def _():
                idx_load(i0 + 3, ib1, semi1)

            idx_wait(ib0, semi0)
            return carry

        lax.fori_loop(0, (nchd - 1) // 2, step, 0)
        scat(ib0)

        plsc.subcore_barrier()

        @pl.when(c == 0)
        def _():
            _copy_out_slices(dacc, st, deg_a, rbase, s)

        @pl.when(c == 1)
        def _():
            _copy_out_slices(dacc, st, deg_b, rbase, s)

    return deg


_AGG = _make_agg()
_DEG = _make_deg()

_BN = 1000


def _make_update(relu, final):
    def body(aa, ab, dga, dgb, ha, hb, wna, wnb, wsa, wsb, bb, *outs):
        degc = dga[...][:, 0:1] + dgb[...][:, 0:1]
        inv = 1.0 / jnp.maximum(degc, 1.0)
        acc = jnp.dot(aa[...] * inv, wna[...],
                      preferred_element_type=jnp.float32)
        acc = acc + jnp.dot(ab[...] * inv, wnb[...],
                            preferred_element_type=jnp.float32)
        acc = acc + jnp.dot(ha[...], wsa[...],
                            preferred_element_type=jnp.float32)
        acc = acc + jnp.dot(hb[...], wsb[...],
                            preferred_element_type=jnp.float32)
        acc = acc + bb[...]
        if relu:
            acc = jnp.maximum(acc, 0.0)
        if final:
            outs[0][...] = acc
        else:
            outs[0][...] = acc[:, :_H]
            outs[1][...] = acc[:, _H:]

    half = pl.BlockSpec((_BN, _H), lambda i: (i, 0))
    in_specs = [
        half, half,
        half, half,
        half, half,
        pl.BlockSpec((_H, _D), lambda i: (0, 0)),
        pl.BlockSpec((_H, _D), lambda i: (0, 0)),
        pl.BlockSpec((_H, _D), lambda i: (0, 0)),
        pl.BlockSpec((_H, _D), lambda i: (0, 0)),
        pl.BlockSpec((1, _D), lambda i: (0, 0)),
    ]
    if final:
        out_specs = pl.BlockSpec((_BN, _D), lambda i: (i, 0))
        out_shape = jax.ShapeDtypeStruct((_N, _D), jnp.float32)
    else:
        out_specs = [half, half]
        out_shape = [jax.ShapeDtypeStruct((_N, _H), jnp.float32),
                     jax.ShapeDtypeStruct((_N, _H), jnp.float32)]
    return pl.pallas_call(body, grid=(_N // _BN,), in_specs=in_specs,
                          out_specs=out_specs, out_shape=out_shape)


_UPDATE_MID = _make_update(True, False)
_UPDATE_FIN = _make_update(False, True)


def _weights(Wn, Ws, b):
    return (Wn[:, :_H].T, Wn[:, _H:].T, Ws[:, :_H].T, Ws[:, _H:].T,
            b.reshape(1, _D))


def kernel(x, edge_index, W_self_0, W_neigh_0, b_0, W_self_1, W_neigh_1,
           b_1, W_self_2, W_neigh_2, b_2):
    edges = jnp.stack(
        [edge_index[0].reshape(_NSUB, _NCH, _K),
         edge_index[1].reshape(_NSUB, _NCH, _K)], axis=2)
    zrows = jnp.zeros((_K, _H), jnp.float32)
    z40 = jnp.zeros((40, _H), jnp.float32)
    ones40 = jnp.ones((40, _H), jnp.float32)
    dst3 = edge_index[1].reshape(2 * _NSUB, _E // 2 // _NSUB // 40, 40)

    deg_a, deg_b = _DEG(dst3, z40, ones40)
    ha, hb = x[:, :_H], x[:, _H:]
    agg_a, agg_b = _AGG(ha, hb, edges, zrows)
    ha, hb = _UPDATE_MID(agg_a, agg_b, deg_a, deg_b, ha, hb,
                         *_weights(W_neigh_0, W_self_0, b_0))
    agg_a, agg_b = _AGG(ha, hb, edges, zrows)
    ha, hb = _UPDATE_MID(agg_a, agg_b, deg_a, deg_b, ha, hb,
                         *_weights(W_neigh_1, W_self_1, b_1))
    agg_a, agg_b = _AGG(ha, hb, edges, zrows)
    return _UPDATE_FIN(agg_a, agg_b, deg_a, deg_b, ha, hb,
                       *_weights(W_neigh_2, W_self_2, b_2))
